# Initial kernel scaffold; baseline (speedup 1.0000x reference)
import jax
import jax.numpy as jnp
from jax.experimental import pallas as pl


def kernel(word, char, word_table, char_table, W):
    raise NotImplementedError("write your pallas kernel here")


# speedup vs baseline: 11.1576x; 11.1576x over previous
import functools

import jax
import jax.numpy as jnp
from jax import lax
from jax.experimental import pallas as pl
from jax.experimental.pallas import tpu as pltpu
from jax.experimental.pallas import tpu_sc as plsc

WORD_VOCAB = 100000
WORD_DIM = 300
CHAR_VOCAB = 10000
EMB_DIM = 128
B, L = 1024, 200
N_TOK = B * L
NC, NS = 2, 16
NW = NC * NS
ROWS_PER_W = N_TOK // NW
CHUNK = 128
N_CHUNKS = ROWS_PER_W // CHUNK
LANES = 16


def _proj_body(wt_ref, w_ref, out_ref):
    out_ref[...] = jax.lax.dot_general(
        wt_ref[...], w_ref[...],
        (((1,), (1,)), ((), ())),
        preferred_element_type=jnp.float32,
    )


def _project(word_table, W):
    BLK = 4000
    return pl.pallas_call(
        _proj_body,
        grid=(WORD_VOCAB // BLK,),
        in_specs=[
            pl.BlockSpec((BLK, WORD_DIM), lambda i: (i, 0)),
            pl.BlockSpec((EMB_DIM, WORD_DIM), lambda i: (0, 0)),
        ],
        out_specs=pl.BlockSpec((BLK, EMB_DIM), lambda i: (i, 0)),
        out_shape=jax.ShapeDtypeStruct((WORD_VOCAB, EMB_DIM), jnp.float32),
    )(word_table, W)


_mesh = plsc.VectorSubcoreMesh(
    core_axis_name="c", subcore_axis_name="s", num_cores=NC, num_subcores=NS
)


@functools.partial(
    pl.kernel,
    out_type=jax.ShapeDtypeStruct((N_TOK, EMB_DIM), jnp.float32),
    mesh=_mesh,
    scratch_types=[
        pltpu.VMEM((1, N_CHUNKS, CHUNK), jnp.int32),
        pltpu.VMEM((1, N_CHUNKS, CHUNK), jnp.int32),
        pltpu.VMEM((CHUNK, EMB_DIM), jnp.float32),
        pltpu.VMEM((CHUNK, EMB_DIM), jnp.float32),
        pltpu.SemaphoreType.DMA,
        pltpu.SemaphoreType.DMA,
    ],
)
def _sc_gather_add(p_hbm, ct_hbm, wi_hbm, ci_hbm, out_hbm,
                   idxw, idxc, acc, crows, sem_a, sem_b):
    wid = lax.axis_index("s") * NC + lax.axis_index("c")
    pltpu.sync_copy(wi_hbm.at[pl.ds(wid, 1)], idxw)
    pltpu.sync_copy(ci_hbm.at[pl.ds(wid, 1)], idxc)

    def chunk(j, carry):
        cp_a = pltpu.async_copy(p_hbm.at[idxw.at[0, j]], acc, sem_a)
        cp_b = pltpu.async_copy(ct_hbm.at[idxc.at[0, j]], crows, sem_b)
        cp_a.wait()
        cp_b.wait()

        def addrow(r, c2):
            for c in range(EMB_DIM // LANES):
                sl = pl.ds(c * LANES, LANES)
                plsc.addupdate(acc.at[r, sl], crows[r, sl])
            return c2

        lax.fori_loop(0, CHUNK, addrow, 0)
        pltpu.sync_copy(
            acc, out_hbm.at[pl.ds(wid * ROWS_PER_W + j * CHUNK, CHUNK)]
        )
        return carry

    lax.fori_loop(0, N_CHUNKS, chunk, 0)


def kernel(word, char, word_table, char_table, W):
    P = _project(word_table, W)
    wi = word.reshape(NW, N_CHUNKS, CHUNK).astype(jnp.int32)
    ci = char.reshape(NW, N_CHUNKS, CHUNK).astype(jnp.int32)
    out = _sc_gather_add(P, char_table, wi, ci)
    return out.reshape(B, L, EMB_DIM)


# speedup vs baseline: 13.8724x; 1.2433x over previous
import functools

import jax
import jax.numpy as jnp
from jax import lax
from jax.experimental import pallas as pl
from jax.experimental.pallas import tpu as pltpu
from jax.experimental.pallas import tpu_sc as plsc

WORD_VOCAB = 100000
WORD_DIM = 300
CHAR_VOCAB = 10000
EMB_DIM = 128
B, L = 1024, 200
N_TOK = B * L
NC, NS = 2, 16
NW = NC * NS
ROWS_PER_W = N_TOK // NW
CHUNK = 128
N_CHUNKS = ROWS_PER_W // CHUNK
LANES = 16


def _proj_body(wt_ref, w_ref, out_ref):
    out_ref[...] = jax.lax.dot_general(
        wt_ref[...], w_ref[...],
        (((1,), (1,)), ((), ())),
        preferred_element_type=jnp.float32,
    )


def _project(word_table, W):
    BLK = 4000
    return pl.pallas_call(
        _proj_body,
        grid=(WORD_VOCAB // BLK,),
        in_specs=[
            pl.BlockSpec((BLK, WORD_DIM), lambda i: (i, 0)),
            pl.BlockSpec((EMB_DIM, WORD_DIM), lambda i: (0, 0)),
        ],
        out_specs=pl.BlockSpec((BLK, EMB_DIM), lambda i: (i, 0)),
        out_shape=jax.ShapeDtypeStruct((WORD_VOCAB, EMB_DIM), jnp.float32),
    )(word_table, W)


_mesh = plsc.VectorSubcoreMesh(
    core_axis_name="c", subcore_axis_name="s", num_cores=NC, num_subcores=NS
)


@functools.partial(
    pl.kernel,
    out_type=jax.ShapeDtypeStruct((N_TOK, EMB_DIM), jnp.float32),
    mesh=_mesh,
    scratch_types=[
        pltpu.VMEM((1, N_CHUNKS, CHUNK), jnp.int32),
        pltpu.VMEM((1, N_CHUNKS, CHUNK), jnp.int32),
        pltpu.VMEM((CHUNK, EMB_DIM), jnp.float32),
        pltpu.VMEM((CHUNK, EMB_DIM), jnp.float32),
        pltpu.VMEM((CHUNK, EMB_DIM), jnp.float32),
        pltpu.VMEM((CHUNK, EMB_DIM), jnp.float32),
        pltpu.VMEM((CHUNK, EMB_DIM), jnp.float32),
        pltpu.VMEM((CHUNK, EMB_DIM), jnp.float32),
        pltpu.SemaphoreType.DMA,
        pltpu.SemaphoreType.DMA,
        pltpu.SemaphoreType.DMA,
        pltpu.SemaphoreType.DMA,
        pltpu.SemaphoreType.DMA,
        pltpu.SemaphoreType.DMA,
    ],
)
def _sc_gather_add(p_hbm, ct_hbm, wi_hbm, ci_hbm, out_hbm,
                   idxw, idxc, a0, a1, b0, b1, o0, o1,
                   sga0, sga1, sgb0, sgb1, ss0, ss1):
    wid = lax.axis_index("s") * NC + lax.axis_index("c")
    pltpu.sync_copy(wi_hbm.at[pl.ds(wid, 1)], idxw)
    pltpu.sync_copy(ci_hbm.at[pl.ds(wid, 1)], idxc)
    base = wid * ROWS_PER_W

    bufs = ((a0, b0, o0, sga0, sgb0, ss0), (a1, b1, o1, sga1, sgb1, ss1))

    def issue_g(j, p):
        a, b, _, sga, sgb, _ = bufs[p]
        pltpu.async_copy(p_hbm.at[idxw.at[0, j]], a, sga)
        pltpu.async_copy(ct_hbm.at[idxc.at[0, j]], b, sgb)

    def wait_g(j, p):
        a, b, _, sga, sgb, _ = bufs[p]
        pltpu.make_async_copy(p_hbm.at[idxw.at[0, j]], a, sga).wait()
        pltpu.make_async_copy(ct_hbm.at[idxc.at[0, j]], b, sgb).wait()

    def issue_s(j, p):
        _, _, o, _, _, ss = bufs[p]
        pltpu.async_copy(o, out_hbm.at[pl.ds(base + j * CHUNK, CHUNK)], ss)

    def wait_s(j, p):
        _, _, o, _, _, ss = bufs[p]
        pltpu.make_async_copy(o, out_hbm.at[pl.ds(base + j * CHUNK, CHUNK)], ss).wait()

    def add_chunk(p):
        a, b, o, _, _, _ = bufs[p]

        def row(r, carry):
            for c in range(EMB_DIM // LANES):
                sl = pl.ds(c * LANES, LANES)
                o[r, sl] = a[r, sl] + b[r, sl]
            return carry

        lax.fori_loop(0, CHUNK, row, 0)

    issue_g(0, 0)
    issue_g(1, 1)

    for p in (0, 1):
        wait_g(p, p)
        add_chunk(p)
        issue_g(p + 2, p)
        issue_s(p, p)

    def steady(g, carry):
        for p in (0, 1):
            j = 2 * g + p
            wait_g(j, p)
            wait_s(j - 2, p)
            add_chunk(p)
            issue_g(j + 2, p)
            issue_s(j, p)
        return carry

    lax.fori_loop(1, N_CHUNKS // 2 - 1, steady, 0)

    for p in (0, 1):
        j = N_CHUNKS - 2 + p
        wait_g(j, p)
        wait_s(j - 2, p)
        add_chunk(p)
        issue_s(j, p)
    for p in (0, 1):
        wait_s(N_CHUNKS - 2 + p, p)


def kernel(word, char, word_table, char_table, W):
    P = _project(word_table, W)
    wi = word.reshape(NW, N_CHUNKS, CHUNK).astype(jnp.int32)
    ci = char.reshape(NW, N_CHUNKS, CHUNK).astype(jnp.int32)
    out = _sc_gather_add(P, char_table, wi, ci)
    return out.reshape(B, L, EMB_DIM)


# speedup vs baseline: 13.8845x; 1.0009x over previous
import functools

import jax
import jax.numpy as jnp
from jax import lax
from jax.experimental import pallas as pl
from jax.experimental.pallas import tpu as pltpu
from jax.experimental.pallas import tpu_sc as plsc

WORD_VOCAB = 100000
WORD_DIM = 300
CHAR_VOCAB = 10000
EMB_DIM = 128
B, L = 1024, 200
N_TOK = B * L
NC, NS = 2, 16
NW = NC * NS
ROWS_PER_W = N_TOK // NW
CHUNK = 128
N_CHUNKS = ROWS_PER_W // CHUNK
LANES = 16
NBUF = 3


def _proj_body(wt_ref, w_ref, out_ref):
    out_ref[...] = jax.lax.dot_general(
        wt_ref[...], w_ref[...],
        (((1,), (1,)), ((), ())),
        preferred_element_type=jnp.float32,
    )


def _project(word_table, W):
    BLK = 4000
    return pl.pallas_call(
        _proj_body,
        grid=(WORD_VOCAB // BLK,),
        in_specs=[
            pl.BlockSpec((BLK, WORD_DIM), lambda i: (i, 0)),
            pl.BlockSpec((EMB_DIM, WORD_DIM), lambda i: (0, 0)),
        ],
        out_specs=pl.BlockSpec((BLK, EMB_DIM), lambda i: (i, 0)),
        out_shape=jax.ShapeDtypeStruct((WORD_VOCAB, EMB_DIM), jnp.float32),
    )(word_table, W)


_mesh = plsc.VectorSubcoreMesh(
    core_axis_name="c", subcore_axis_name="s", num_cores=NC, num_subcores=NS
)

_scratch = (
    [pltpu.VMEM((1, N_CHUNKS, CHUNK), jnp.int32)] * 2
    + [pltpu.VMEM((CHUNK, EMB_DIM), jnp.float32)] * (2 * NBUF)
    + [pltpu.SemaphoreType.DMA] * (3 * NBUF)
)


@functools.partial(
    pl.kernel,
    out_type=jax.ShapeDtypeStruct((N_TOK, EMB_DIM), jnp.float32),
    mesh=_mesh,
    scratch_types=_scratch,
)
def _sc_gather_add(p_hbm, ct_hbm, wi_hbm, ci_hbm, out_hbm,
                   idxw, idxc, o0, o1, o2, b0, b1, b2,
                   sgw0, sgw1, sgw2, sgc0, sgc1, sgc2, ss0, ss1, ss2):
    wid = lax.axis_index("s") * NC + lax.axis_index("c")
    pltpu.sync_copy(wi_hbm.at[pl.ds(wid, 1)], idxw)
    pltpu.sync_copy(ci_hbm.at[pl.ds(wid, 1)], idxc)
    base = wid * ROWS_PER_W

    obuf = (o0, o1, o2)
    bbuf = (b0, b1, b2)
    sgw = (sgw0, sgw1, sgw2)
    sgc = (sgc0, sgc1, sgc2)
    ss = (ss0, ss1, ss2)

    def issue_g(j, p):
        pltpu.async_copy(p_hbm.at[idxw.at[0, j]], obuf[p], sgw[p])
        pltpu.async_copy(ct_hbm.at[idxc.at[0, j]], bbuf[p], sgc[p])

    def wait_g(j, p):
        pltpu.make_async_copy(p_hbm.at[idxw.at[0, j]], obuf[p], sgw[p]).wait()
        pltpu.make_async_copy(ct_hbm.at[idxc.at[0, j]], bbuf[p], sgc[p]).wait()

    def issue_s(j, p):
        pltpu.async_copy(obuf[p], out_hbm.at[pl.ds(base + j * CHUNK, CHUNK)], ss[p])

    def wait_s(j, p):
        pltpu.make_async_copy(
            obuf[p], out_hbm.at[pl.ds(base + j * CHUNK, CHUNK)], ss[p]
        ).wait()

    def add_chunk(p):
        o, b = obuf[p], bbuf[p]

        def row(r, carry):
            for c in range(EMB_DIM // LANES):
                sl = pl.ds(c * LANES, LANES)
                plsc.addupdate(o.at[r, sl], b[r, sl])
            return carry

        lax.fori_loop(0, CHUNK, row, 0)

    def step(j, p, do_wait_s, gnext):
        q = (p + 2) % NBUF
        wait_g(j, p)
        add_chunk(p)
        issue_s(j, p)
        if do_wait_s:
            wait_s(j - 1, q)
        if gnext:
            issue_g(j + 2, q)

    for p in range(NBUF):
        issue_g(p, p)

    step(0, 0, False, False)
    step(1, 1, True, True)
    step(2, 2, True, True)

    def steady(g, carry):
        for k in range(NBUF):
            j = NBUF * g + k
            step(j, k, True, True)
        return carry

    lax.fori_loop(1, 15, steady, 0)

    step(45, 0, True, True)
    step(46, 1, True, True)
    step(47, 2, True, True)
    step(48, 0, True, False)
    step(49, 1, True, False)
    wait_s(49, 1)


def kernel(word, char, word_table, char_table, W):
    P = _project(word_table, W)
    wi = word.reshape(NW, N_CHUNKS, CHUNK).astype(jnp.int32)
    ci = char.reshape(NW, N_CHUNKS, CHUNK).astype(jnp.int32)
    out = _sc_gather_add(P, char_table, wi, ci)
    return out.reshape(B, L, EMB_DIM)
